# baseline (speedup 1.0000x reference)
import functools

import jax
import jax.numpy as jnp
from jax import lax
from jax.experimental import pallas as pl
from jax.experimental.pallas import tpu as pltpu
from jax.experimental.pallas import tpu_sc as plsc

_N = 10000
_E = 320000
_G = 64

_NC = 2
_NS = 16
_NW = _NC * _NS

_CH = 128
_CB = 8
_EPW_PAD = -(-(_E // _NW) // (_CH * _CB)) * _CH * _CB
_NCHUNK = _EPW_PAD // _CH
_NSUP = _NCHUNK // _CB

_NPAD = 10240
_RPT = _NPAD // _NS
_BLK = 1280
_GRID = _NPAD // _BLK


def _wid(cid, sid):
    return cid * _NS + sid


def _deg_body(dstp, out, dst_v, ones_v, zbuf_v, deg_sh):
    cid = lax.axis_index("c")
    sid = lax.axis_index("s")

    ones16 = jnp.full((16,), 1.0, jnp.float32)

    def ob(i, _):
        ones_v[pl.ds(i * 16, 16)] = ones16
        return ()
    lax.fori_loop(0, _NCHUNK * _CH // 16, ob, ())
    zeros16 = jnp.zeros((16,), jnp.float32)

    def zb(i, _):
        zbuf_v[pl.ds(i * 16, 16)] = zeros16
        return ()
    lax.fori_loop(0, _RPT // 16, zb, ())
    pltpu.sync_copy(zbuf_v, deg_sh.at[pl.ds(sid * _RPT, _RPT)])
    plsc.subcore_barrier()

    pltpu.sync_copy(dstp.at[_wid(cid, sid)], dst_v)
    pltpu.sync_copy(ones_v, deg_sh.at[dst_v], add=True)
    plsc.subcore_barrier()

    pltpu.sync_copy(deg_sh.at[pl.ds(sid * _RPT, _RPT)],
                    out.at[cid, pl.ds(sid * _RPT, _RPT)])


def _deg_sc(dstp):
    mesh = plsc.VectorSubcoreMesh(core_axis_name="c", subcore_axis_name="s")
    return pl.kernel(
        _deg_body,
        out_type=jax.ShapeDtypeStruct((_NC, _NPAD), jnp.float32),
        mesh=mesh,
        compiler_params=pltpu.CompilerParams(use_tc_tiling_on_sc=False),
        scratch_types=[
            pltpu.VMEM((_NCHUNK * _CH,), jnp.int32),
            pltpu.VMEM((_NCHUNK * _CH,), jnp.float32),
            pltpu.VMEM((_RPT,), jnp.float32),
            pltpu.VMEM_SHARED((_NPAD,), jnp.float32),
        ],
    )(dstp)


def _conv_body(g, srcp, dstp, out, src_v, dst_v, rows_v, zbuf_v, acc_sh):
    cid = lax.axis_index("c")
    sid = lax.axis_index("s")

    zeros16 = jnp.zeros((16,), jnp.float32)

    def zb(i, _):
        for j in range(4):
            zbuf_v[i, pl.ds(j * 16, 16)] = zeros16
        return ()
    lax.fori_loop(0, 64, zb, ())
    for r in range(_RPT // 64):
        pltpu.sync_copy(zbuf_v, acc_sh.at[pl.ds(sid * _RPT + r * 64, 64)])
    plsc.subcore_barrier()

    w = _wid(cid, sid)
    pltpu.sync_copy(srcp.at[w], src_v)
    pltpu.sync_copy(dstp.at[w], dst_v)
    sup = _CB * _CH

    def step(j, _):
        pltpu.sync_copy(g.at[src_v.at[pl.ds(j * sup, sup)]], rows_v)
        pltpu.sync_copy(rows_v, acc_sh.at[dst_v.at[pl.ds(j * sup, sup)]],
                        add=True)
        return ()
    lax.fori_loop(0, _NSUP, step, ())
    plsc.subcore_barrier()

    pltpu.sync_copy(acc_sh.at[pl.ds(sid * _RPT, _RPT)],
                    out.at[cid, pl.ds(sid * _RPT, _RPT)])


def _conv_sc(g, srcp, dstp):
    mesh = plsc.VectorSubcoreMesh(core_axis_name="c", subcore_axis_name="s")
    return pl.kernel(
        _conv_body,
        out_type=jax.ShapeDtypeStruct((_NC, _NPAD, 64), jnp.float32),
        mesh=mesh,
        compiler_params=pltpu.CompilerParams(use_tc_tiling_on_sc=False),
        scratch_types=[
            pltpu.VMEM((_NCHUNK * _CH,), jnp.int32),
            pltpu.VMEM((_NCHUNK * _CH,), jnp.int32),
            pltpu.VMEM((_CB * _CH, 64), jnp.float32),
            pltpu.VMEM((64, 64), jnp.float32),
            pltpu.VMEM_SHARED((_NPAD, 64), jnp.float32),
        ],
    )(g, srcp, dstp)


def _dinv(dpt_ref):
    deg = dpt_ref[:, 0:1] + dpt_ref[:, 1:2] + 1.0
    return lax.rsqrt(deg)


def _mm1_body(x_ref, w_ref, dpt_ref, g_ref):
    h = jnp.dot(x_ref[...], w_ref[...], preferred_element_type=jnp.float32)
    g_ref[...] = _dinv(dpt_ref) * h


def _mid_body(ap_ref, g1_ref, dpt_ref, b1_ref, w2_ref, g2_ref):
    dinv = _dinv(dpt_ref)
    acc = ap_ref[0] + ap_ref[1] + g1_ref[...]
    h1 = jnp.maximum(dinv * acc + b1_ref[...], 0.0)
    g2_ref[...] = dinv * jnp.dot(h1, w2_ref[...],
                                 preferred_element_type=jnp.float32)


def _pool_body(ap_ref, g2_ref, dpt_ref, b2_ref, bat_ref, out_ref):
    i = pl.program_id(0)
    dinv = _dinv(dpt_ref)
    h2 = dinv * (ap_ref[0] + ap_ref[1] + g2_ref[...]) + b2_ref[...]
    ids = jax.lax.broadcasted_iota(jnp.int32, (_G, _BLK), 0)
    oht = (ids == bat_ref[0]).astype(jnp.float32)
    part = jnp.dot(oht, h2, preferred_element_type=jnp.float32)

    @pl.when(i == 0)
    def _():
        out_ref[...] = part

    @pl.when(i > 0)
    def _():
        out_ref[...] += part


def _mm1_tc(xp, W1, dpt):
    return pl.pallas_call(
        _mm1_body,
        grid=(_GRID,),
        in_specs=[pl.BlockSpec((_BLK, 128), lambda i: (i, 0)),
                  pl.BlockSpec((128, 64), lambda i: (0, 0)),
                  pl.BlockSpec((_BLK, _NC), lambda i: (i, 0))],
        out_specs=pl.BlockSpec((_BLK, 64), lambda i: (i, 0)),
        out_shape=jax.ShapeDtypeStruct((_NPAD, 64), jnp.float32),
    )(xp, W1, dpt)


def _mid_tc(ap, g1, dpt, b1, W2):
    return pl.pallas_call(
        _mid_body,
        grid=(_GRID,),
        in_specs=[pl.BlockSpec((_NC, _BLK, 64), lambda i: (0, i, 0)),
                  pl.BlockSpec((_BLK, 64), lambda i: (i, 0)),
                  pl.BlockSpec((_BLK, _NC), lambda i: (i, 0)),
                  pl.BlockSpec((1, 64), lambda i: (0, 0)),
                  pl.BlockSpec((64, 64), lambda i: (0, 0))],
        out_specs=pl.BlockSpec((_BLK, 64), lambda i: (i, 0)),
        out_shape=jax.ShapeDtypeStruct((_NPAD, 64), jnp.float32),
    )(ap, g1, dpt, b1, W2)


def _pool_tc(ap, g2, dpt, b2, bat3):
    return pl.pallas_call(
        _pool_body,
        grid=(_GRID,),
        in_specs=[pl.BlockSpec((_NC, _BLK, 64), lambda i: (0, i, 0)),
                  pl.BlockSpec((_BLK, 64), lambda i: (i, 0)),
                  pl.BlockSpec((_BLK, _NC), lambda i: (i, 0)),
                  pl.BlockSpec((1, 64), lambda i: (0, 0)),
                  pl.BlockSpec((1, 1, _BLK), lambda i: (i, 0, 0))],
        out_specs=pl.BlockSpec((_G, 64), lambda i: (0, 0)),
        out_shape=jax.ShapeDtypeStruct((_G, 64), jnp.float32),
    )(ap, g2, dpt, b2, bat3)


def kernel(x, edge_index, batch, W1, b1, W2, b2):
    src, dst = edge_index[0], edge_index[1]
    epw = _E // _NW
    pad = _EPW_PAD - epw
    srcp = jnp.pad(src.reshape(_NW, epw), ((0, 0), (0, pad)))
    dstp = jnp.pad(dst.reshape(_NW, epw), ((0, 0), (0, pad)),
                   constant_values=_N)

    xp = jnp.pad(x, ((0, _NPAD - _N), (0, 0)))
    bat3 = jnp.pad(batch, (0, _NPAD - _N),
                   constant_values=_G).reshape(_GRID, 1, _BLK)

    dp = _deg_sc(dstp)
    dpt = dp.T

    g1 = _mm1_tc(xp, W1, dpt)
    ap1 = _conv_sc(g1, srcp, dstp)
    g2 = _mid_tc(ap1, g1, dpt, b1.reshape(1, 64), W2)
    ap2 = _conv_sc(g2, srcp, dstp)
    out = _pool_tc(ap2, g2, dpt, b2.reshape(1, 64), bat3)
    return out


# speedup vs baseline: 1.0804x; 1.0804x over previous
import functools

import jax
import jax.numpy as jnp
from jax import lax
from jax.experimental import pallas as pl
from jax.experimental.pallas import tpu as pltpu
from jax.experimental.pallas import tpu_sc as plsc

_N = 10000
_E = 320000
_G = 64

_NC = 2
_NS = 16
_NW = _NC * _NS

_CH = 128
_CB = 4
_EPW_PAD = -(-(_E // _NW) // (_CH * _CB)) * _CH * _CB
_NCHUNK = _EPW_PAD // _CH
_NSUP = _NCHUNK // _CB

_NPAD = 10240
_RPT = _NPAD // _NS
_BLK = 1280
_GRID = _NPAD // _BLK


def _wid(cid, sid):
    return cid * _NS + sid


def _deg_body(dstp, out, dst_v, ones_v, zbuf_v, deg_sh):
    cid = lax.axis_index("c")
    sid = lax.axis_index("s")

    ones16 = jnp.full((16,), 1.0, jnp.float32)

    def ob(i, _):
        ones_v[pl.ds(i * 16, 16)] = ones16
        return ()
    lax.fori_loop(0, _NCHUNK * _CH // 16, ob, ())
    zeros16 = jnp.zeros((16,), jnp.float32)

    def zb(i, _):
        zbuf_v[pl.ds(i * 16, 16)] = zeros16
        return ()
    lax.fori_loop(0, _RPT // 16, zb, ())
    pltpu.sync_copy(zbuf_v, deg_sh.at[pl.ds(sid * _RPT, _RPT)])
    plsc.subcore_barrier()

    pltpu.sync_copy(dstp.at[_wid(cid, sid)], dst_v)
    pltpu.sync_copy(ones_v, deg_sh.at[dst_v], add=True)
    plsc.subcore_barrier()

    pltpu.sync_copy(deg_sh.at[pl.ds(sid * _RPT, _RPT)],
                    out.at[cid, pl.ds(sid * _RPT, _RPT)])


def _deg_sc(dstp):
    mesh = plsc.VectorSubcoreMesh(core_axis_name="c", subcore_axis_name="s")
    return pl.kernel(
        _deg_body,
        out_type=jax.ShapeDtypeStruct((_NC, _NPAD), jnp.float32),
        mesh=mesh,
        compiler_params=pltpu.CompilerParams(use_tc_tiling_on_sc=False),
        scratch_types=[
            pltpu.VMEM((_NCHUNK * _CH,), jnp.int32),
            pltpu.VMEM((_NCHUNK * _CH,), jnp.float32),
            pltpu.VMEM((_RPT,), jnp.float32),
            pltpu.VMEM_SHARED((_NPAD,), jnp.float32),
        ],
    )(dstp)


def _conv_body(g, srcp, dstp, out, src_v, dst_v, rows0_v, rows1_v, zbuf_v,
               acc_sh, sg0, sg1, ss0, ss1):
    cid = lax.axis_index("c")
    sid = lax.axis_index("s")

    zeros16 = jnp.zeros((16,), jnp.float32)

    def zb(i, _):
        for j in range(4):
            zbuf_v[i, pl.ds(j * 16, 16)] = zeros16
        return ()
    lax.fori_loop(0, 64, zb, ())
    for r in range(_RPT // 64):
        pltpu.sync_copy(zbuf_v, acc_sh.at[pl.ds(sid * _RPT + r * 64, 64)])
    plsc.subcore_barrier()

    w = _wid(cid, sid)
    pltpu.sync_copy(srcp.at[w], src_v)
    pltpu.sync_copy(dstp.at[w], dst_v)
    sup = _CB * _CH

    bufs = (rows0_v, rows1_v)
    gsem = (sg0, sg1)
    ssem = (ss0, ss1)

    def start_g(j, b):
        return pltpu.async_copy(g.at[src_v.at[pl.ds(j * sup, sup)]],
                                bufs[b], gsem[b])

    def start_s(j, b):
        return pltpu.async_copy(bufs[b],
                                acc_sh.at[dst_v.at[pl.ds(j * sup, sup)]],
                                ssem[b], add=True)

    gh = [None, None]
    sh = [None, None]
    gh[0] = start_g(0, 0)
    for j in range(_NSUP):
        b = j & 1
        if j + 1 < _NSUP:
            if sh[1 - b] is not None:
                sh[1 - b].wait()
            gh[1 - b] = start_g(j + 1, 1 - b)
        gh[b].wait()
        sh[b] = start_s(j, b)
    sh[(_NSUP - 2) & 1].wait()
    sh[(_NSUP - 1) & 1].wait()
    plsc.subcore_barrier()

    pltpu.sync_copy(acc_sh.at[pl.ds(sid * _RPT, _RPT)],
                    out.at[cid, pl.ds(sid * _RPT, _RPT)])


def _conv_sc(g, srcp, dstp):
    mesh = plsc.VectorSubcoreMesh(core_axis_name="c", subcore_axis_name="s")
    return pl.kernel(
        _conv_body,
        out_type=jax.ShapeDtypeStruct((_NC, _NPAD, 64), jnp.float32),
        mesh=mesh,
        compiler_params=pltpu.CompilerParams(use_tc_tiling_on_sc=False),
        scratch_types=[
            pltpu.VMEM((_NCHUNK * _CH,), jnp.int32),
            pltpu.VMEM((_NCHUNK * _CH,), jnp.int32),
            pltpu.VMEM((_CB * _CH, 64), jnp.float32),
            pltpu.VMEM((_CB * _CH, 64), jnp.float32),
            pltpu.VMEM((64, 64), jnp.float32),
            pltpu.VMEM_SHARED((_NPAD, 64), jnp.float32),
            pltpu.SemaphoreType.DMA,
            pltpu.SemaphoreType.DMA,
            pltpu.SemaphoreType.DMA,
            pltpu.SemaphoreType.DMA,
        ],
    )(g, srcp, dstp)


def _dinv(dpt_ref):
    deg = dpt_ref[:, 0:1] + dpt_ref[:, 1:2] + 1.0
    return lax.rsqrt(deg)


def _mm1_body(x_ref, w_ref, dpt_ref, g_ref):
    h = jnp.dot(x_ref[...], w_ref[...], preferred_element_type=jnp.float32)
    g_ref[...] = _dinv(dpt_ref) * h


def _mid_body(ap_ref, g1_ref, dpt_ref, b1_ref, w2_ref, g2_ref):
    dinv = _dinv(dpt_ref)
    acc = ap_ref[0] + ap_ref[1] + g1_ref[...]
    h1 = jnp.maximum(dinv * acc + b1_ref[...], 0.0)
    g2_ref[...] = dinv * jnp.dot(h1, w2_ref[...],
                                 preferred_element_type=jnp.float32)


def _pool_body(ap_ref, g2_ref, dpt_ref, b2_ref, bat_ref, out_ref):
    i = pl.program_id(0)
    dinv = _dinv(dpt_ref)
    h2 = dinv * (ap_ref[0] + ap_ref[1] + g2_ref[...]) + b2_ref[...]
    ids = jax.lax.broadcasted_iota(jnp.int32, (_G, _BLK), 0)
    oht = (ids == bat_ref[0]).astype(jnp.float32)
    part = jnp.dot(oht, h2, preferred_element_type=jnp.float32)

    @pl.when(i == 0)
    def _():
        out_ref[...] = part

    @pl.when(i > 0)
    def _():
        out_ref[...] += part


def _mm1_tc(xp, W1, dpt):
    return pl.pallas_call(
        _mm1_body,
        grid=(_GRID,),
        in_specs=[pl.BlockSpec((_BLK, 128), lambda i: (i, 0)),
                  pl.BlockSpec((128, 64), lambda i: (0, 0)),
                  pl.BlockSpec((_BLK, _NC), lambda i: (i, 0))],
        out_specs=pl.BlockSpec((_BLK, 64), lambda i: (i, 0)),
        out_shape=jax.ShapeDtypeStruct((_NPAD, 64), jnp.float32),
    )(xp, W1, dpt)


def _mid_tc(ap, g1, dpt, b1, W2):
    return pl.pallas_call(
        _mid_body,
        grid=(_GRID,),
        in_specs=[pl.BlockSpec((_NC, _BLK, 64), lambda i: (0, i, 0)),
                  pl.BlockSpec((_BLK, 64), lambda i: (i, 0)),
                  pl.BlockSpec((_BLK, _NC), lambda i: (i, 0)),
                  pl.BlockSpec((1, 64), lambda i: (0, 0)),
                  pl.BlockSpec((64, 64), lambda i: (0, 0))],
        out_specs=pl.BlockSpec((_BLK, 64), lambda i: (i, 0)),
        out_shape=jax.ShapeDtypeStruct((_NPAD, 64), jnp.float32),
    )(ap, g1, dpt, b1, W2)


def _pool_tc(ap, g2, dpt, b2, bat3):
    return pl.pallas_call(
        _pool_body,
        grid=(_GRID,),
        in_specs=[pl.BlockSpec((_NC, _BLK, 64), lambda i: (0, i, 0)),
                  pl.BlockSpec((_BLK, 64), lambda i: (i, 0)),
                  pl.BlockSpec((_BLK, _NC), lambda i: (i, 0)),
                  pl.BlockSpec((1, 64), lambda i: (0, 0)),
                  pl.BlockSpec((1, 1, _BLK), lambda i: (i, 0, 0))],
        out_specs=pl.BlockSpec((_G, 64), lambda i: (0, 0)),
        out_shape=jax.ShapeDtypeStruct((_G, 64), jnp.float32),
    )(ap, g2, dpt, b2, bat3)


def kernel(x, edge_index, batch, W1, b1, W2, b2):
    src, dst = edge_index[0], edge_index[1]
    epw = _E // _NW
    pad = _EPW_PAD - epw
    srcp = jnp.pad(src.reshape(_NW, epw), ((0, 0), (0, pad)))
    dstp = jnp.pad(dst.reshape(_NW, epw), ((0, 0), (0, pad)),
                   constant_values=_N)

    xp = jnp.pad(x, ((0, _NPAD - _N), (0, 0)))
    bat3 = jnp.pad(batch, (0, _NPAD - _N),
                   constant_values=_G).reshape(_GRID, 1, _BLK)

    dp = _deg_sc(dstp)
    dpt = dp.T

    g1 = _mm1_tc(xp, W1, dpt)
    ap1 = _conv_sc(g1, srcp, dstp)
    g2 = _mid_tc(ap1, g1, dpt, b1.reshape(1, 64), W2)
    ap2 = _conv_sc(g2, srcp, dstp)
    out = _pool_tc(ap2, g2, dpt, b2.reshape(1, 64), bat3)
    return out


# speedup vs baseline: 1.5914x; 1.4730x over previous
import functools

import jax
import jax.numpy as jnp
from jax import lax
from jax.experimental import pallas as pl
from jax.experimental.pallas import tpu as pltpu
from jax.experimental.pallas import tpu_sc as plsc

_N = 10000
_E = 320000
_G = 64

_NC = 2
_NS = 16
_NW = _NC * _NS

_CH = 128
_EPW_PAD = -(-(_E // _NW) // _CH) * _CH
_NCHUNK = _EPW_PAD // _CH

_NPAD = 10240
_RPT = _NPAD // _NS
_BLK = 1280
_GRID = _NPAD // _BLK


def _wid(cid, sid):
    return cid * _NS + sid


def _deg_body(dstp, out, dst_v, ones_v, zbuf_v, deg_sh):
    cid = lax.axis_index("c")
    sid = lax.axis_index("s")

    ones16 = jnp.full((16,), 1.0, jnp.float32)

    def ob(i, _):
        ones_v[pl.ds(i * 16, 16)] = ones16
        return ()
    lax.fori_loop(0, _EPW_PAD // 16, ob, ())
    zeros16 = jnp.zeros((16,), jnp.float32)

    def zb(i, _):
        zbuf_v[pl.ds(i * 16, 16)] = zeros16
        return ()
    lax.fori_loop(0, _RPT // 16, zb, ())
    pltpu.sync_copy(zbuf_v, deg_sh.at[pl.ds(sid * _RPT, _RPT)])
    plsc.subcore_barrier()

    pltpu.sync_copy(dstp.at[_wid(cid, sid)], dst_v)
    pltpu.sync_copy(ones_v, deg_sh.at[dst_v], add=True)
    plsc.subcore_barrier()

    pltpu.sync_copy(deg_sh.at[pl.ds(sid * _RPT, _RPT)],
                    out.at[cid, pl.ds(sid * _RPT, _RPT)])


def _deg_sc(dstf):
    mesh = plsc.VectorSubcoreMesh(core_axis_name="c", subcore_axis_name="s")
    return pl.kernel(
        _deg_body,
        out_type=jax.ShapeDtypeStruct((_NC, _NPAD), jnp.float32),
        mesh=mesh,
        compiler_params=pltpu.CompilerParams(use_tc_tiling_on_sc=False),
        scratch_types=[
            pltpu.VMEM((_EPW_PAD,), jnp.int32),
            pltpu.VMEM((_EPW_PAD,), jnp.float32),
            pltpu.VMEM((_RPT,), jnp.float32),
            pltpu.VMEM_SHARED((_NPAD,), jnp.float32),
        ],
    )(dstf)


def _conv_body(g, srcp, dstp, out, src_v, dst_v, rows_v, zbuf_v, acc_sh):
    cid = lax.axis_index("c")
    sid = lax.axis_index("s")

    zeros32 = jnp.zeros((32,), jnp.bfloat16)

    def zb(i, _):
        for j in range(2):
            zbuf_v[i, pl.ds(j * 32, 32)] = zeros32
        return ()
    lax.fori_loop(0, 64, zb, ())
    for r in range(_RPT // 64):
        pltpu.sync_copy(zbuf_v, acc_sh.at[pl.ds(sid * _RPT + r * 64, 64)])
    plsc.subcore_barrier()

    w = _wid(cid, sid)
    pltpu.sync_copy(srcp.at[w], src_v)
    pltpu.sync_copy(dstp.at[w], dst_v)

    def step(j, _):
        pltpu.sync_copy(g.at[src_v.at[j]], rows_v)
        pltpu.sync_copy(rows_v, acc_sh.at[dst_v.at[j]], add=True)
        return ()
    lax.fori_loop(0, _NCHUNK, step, ())
    plsc.subcore_barrier()

    pltpu.sync_copy(acc_sh.at[pl.ds(sid * _RPT, _RPT)],
                    out.at[cid, pl.ds(sid * _RPT, _RPT)])


def _conv_sc(g, srcp, dstp):
    mesh = plsc.VectorSubcoreMesh(core_axis_name="c", subcore_axis_name="s")
    return pl.kernel(
        _conv_body,
        out_type=jax.ShapeDtypeStruct((_NC, _NPAD, 64), jnp.bfloat16),
        mesh=mesh,
        compiler_params=pltpu.CompilerParams(use_tc_tiling_on_sc=False),
        scratch_types=[
            pltpu.VMEM((_NCHUNK, _CH), jnp.int32),
            pltpu.VMEM((_NCHUNK, _CH), jnp.int32),
            pltpu.VMEM((_CH, 64), jnp.bfloat16),
            pltpu.VMEM((64, 64), jnp.bfloat16),
            pltpu.VMEM_SHARED((_NPAD, 64), jnp.bfloat16),
        ],
    )(g, srcp, dstp)


def _dinv(dpt_ref):
    deg = dpt_ref[:, 0:1] + dpt_ref[:, 1:2] + 1.0
    return lax.rsqrt(deg)


def _mm1_body(x_ref, w_ref, dpt_ref, g_ref):
    h = jnp.dot(x_ref[...], w_ref[...], preferred_element_type=jnp.float32)
    g_ref[...] = (_dinv(dpt_ref) * h).astype(jnp.bfloat16)


def _mid_body(ap_ref, g1_ref, dpt_ref, b1_ref, w2_ref, g2_ref):
    dinv = _dinv(dpt_ref)
    acc = (ap_ref[0] + ap_ref[1]).astype(jnp.float32) \
        + g1_ref[...].astype(jnp.float32)
    h1 = jnp.maximum(dinv * acc + b1_ref[...], 0.0)
    g2_ref[...] = (dinv * jnp.dot(h1, w2_ref[...],
                                  preferred_element_type=jnp.float32)
                   ).astype(jnp.bfloat16)


def _pool_body(ap_ref, g2_ref, dpt_ref, b2_ref, bat_ref, out_ref):
    i = pl.program_id(0)
    dinv = _dinv(dpt_ref)
    h2 = dinv * ((ap_ref[0] + ap_ref[1]).astype(jnp.float32)
                 + g2_ref[...].astype(jnp.float32)) + b2_ref[...]
    ids = jax.lax.broadcasted_iota(jnp.int32, (_G, _BLK), 0)
    oht = (ids == bat_ref[0]).astype(jnp.float32)
    part = jnp.dot(oht, h2, preferred_element_type=jnp.float32)

    @pl.when(i == 0)
    def _():
        out_ref[...] = part

    @pl.when(i > 0)
    def _():
        out_ref[...] += part


def _mm1_tc(xp, W1, dpt):
    return pl.pallas_call(
        _mm1_body,
        grid=(_GRID,),
        in_specs=[pl.BlockSpec((_BLK, 128), lambda i: (i, 0)),
                  pl.BlockSpec((128, 64), lambda i: (0, 0)),
                  pl.BlockSpec((_BLK, _NC), lambda i: (i, 0))],
        out_specs=pl.BlockSpec((_BLK, 64), lambda i: (i, 0)),
        out_shape=jax.ShapeDtypeStruct((_NPAD, 64), jnp.bfloat16),
    )(xp, W1, dpt)


def _mid_tc(ap, g1, dpt, b1, W2):
    return pl.pallas_call(
        _mid_body,
        grid=(_GRID,),
        in_specs=[pl.BlockSpec((_NC, _BLK, 64), lambda i: (0, i, 0)),
                  pl.BlockSpec((_BLK, 64), lambda i: (i, 0)),
                  pl.BlockSpec((_BLK, _NC), lambda i: (i, 0)),
                  pl.BlockSpec((1, 64), lambda i: (0, 0)),
                  pl.BlockSpec((64, 64), lambda i: (0, 0))],
        out_specs=pl.BlockSpec((_BLK, 64), lambda i: (i, 0)),
        out_shape=jax.ShapeDtypeStruct((_NPAD, 64), jnp.bfloat16),
    )(ap, g1, dpt, b1, W2)


def _pool_tc(ap, g2, dpt, b2, bat3):
    return pl.pallas_call(
        _pool_body,
        grid=(_GRID,),
        in_specs=[pl.BlockSpec((_NC, _BLK, 64), lambda i: (0, i, 0)),
                  pl.BlockSpec((_BLK, 64), lambda i: (i, 0)),
                  pl.BlockSpec((_BLK, _NC), lambda i: (i, 0)),
                  pl.BlockSpec((1, 64), lambda i: (0, 0)),
                  pl.BlockSpec((1, 1, _BLK), lambda i: (i, 0, 0))],
        out_specs=pl.BlockSpec((_G, 64), lambda i: (0, 0)),
        out_shape=jax.ShapeDtypeStruct((_G, 64), jnp.float32),
    )(ap, g2, dpt, b2, bat3)


def kernel(x, edge_index, batch, W1, b1, W2, b2):
    src, dst = edge_index[0], edge_index[1]
    epw = _E // _NW
    pad = _EPW_PAD - epw
    srcp = jnp.pad(src.reshape(_NW, epw), ((0, 0), (0, pad))
                   ).reshape(_NW, _NCHUNK, _CH)
    dstf = jnp.pad(dst.reshape(_NW, epw), ((0, 0), (0, pad)),
                   constant_values=_N)
    dstp = dstf.reshape(_NW, _NCHUNK, _CH)

    xp = jnp.pad(x, ((0, _NPAD - _N), (0, 0)))
    bat3 = jnp.pad(batch, (0, _NPAD - _N),
                   constant_values=_G).reshape(_GRID, 1, _BLK)

    dp = _deg_sc(dstf)
    dpt = dp.T

    g1 = _mm1_tc(xp, W1, dpt)
    ap1 = _conv_sc(g1, srcp, dstp)
    g2 = _mid_tc(ap1, g1, dpt, b1.reshape(1, 64), W2)
    ap2 = _conv_sc(g2, srcp, dstp)
    out = _pool_tc(ap2, g2, dpt, b2.reshape(1, 64), bat3)
    return out


# speedup vs baseline: 2.4409x; 1.5338x over previous
import functools

import jax
import jax.numpy as jnp
from jax import lax
from jax.experimental import pallas as pl
from jax.experimental.pallas import tpu as pltpu
from jax.experimental.pallas import tpu_sc as plsc

_N = 10000
_E = 320000
_G = 64

_NC = 2
_NS = 16
_NW = _NC * _NS

_CH = 128
_EPW_PAD = -(-(_E // _NW) // _CH) * _CH
_NCHUNK = _EPW_PAD // _CH

_NPAD = 10240
_RPT = _NPAD // _NS
_BLK = 1280
_GRID = _NPAD // _BLK


def _wid(cid, sid):
    return cid * _NS + sid


def _deg_body(dstp, out, dst_v, ones_v, zbuf_v, deg_sh):
    cid = lax.axis_index("c")
    sid = lax.axis_index("s")

    ones16 = jnp.full((16,), 1.0, jnp.float32)

    def ob(i, _):
        ones_v[pl.ds(i * 16, 16)] = ones16
        return ()
    lax.fori_loop(0, _EPW_PAD // 16, ob, ())
    zeros16 = jnp.zeros((16,), jnp.float32)

    def zb(i, _):
        zbuf_v[pl.ds(i * 16, 16)] = zeros16
        return ()
    lax.fori_loop(0, _RPT // 16, zb, ())
    pltpu.sync_copy(zbuf_v, deg_sh.at[pl.ds(sid * _RPT, _RPT)])
    plsc.subcore_barrier()

    pltpu.sync_copy(dstp.at[_wid(cid, sid)], dst_v)
    pltpu.sync_copy(ones_v, deg_sh.at[dst_v], add=True)
    plsc.subcore_barrier()

    pltpu.sync_copy(deg_sh.at[pl.ds(sid * _RPT, _RPT)],
                    out.at[cid, pl.ds(sid * _RPT, _RPT)])


def _deg_sc(dstf):
    mesh = plsc.VectorSubcoreMesh(core_axis_name="c", subcore_axis_name="s")
    return pl.kernel(
        _deg_body,
        out_type=jax.ShapeDtypeStruct((_NC, _NPAD), jnp.float32),
        mesh=mesh,
        compiler_params=pltpu.CompilerParams(use_tc_tiling_on_sc=False),
        scratch_types=[
            pltpu.VMEM((_EPW_PAD,), jnp.int32),
            pltpu.VMEM((_EPW_PAD,), jnp.float32),
            pltpu.VMEM((_RPT,), jnp.float32),
            pltpu.VMEM_SHARED((_NPAD,), jnp.float32),
        ],
    )(dstf)


def _conv_body(g, srcp, dstp, out, src_v, dst_v, rows_v, zbuf_v, acc_sh,
               g_sh):
    cid = lax.axis_index("c")
    sid = lax.axis_index("s")

    zeros32 = jnp.zeros((32,), jnp.bfloat16)

    def zb(i, _):
        for j in range(2):
            zbuf_v[i, pl.ds(j * 32, 32)] = zeros32
        return ()
    lax.fori_loop(0, 64, zb, ())
    pltpu.sync_copy(g.at[pl.ds(sid * _RPT, _RPT)],
                    g_sh.at[pl.ds(sid * _RPT, _RPT)])
    for r in range(_RPT // 64):
        pltpu.sync_copy(zbuf_v, acc_sh.at[pl.ds(sid * _RPT + r * 64, 64)])
    plsc.subcore_barrier()

    w = _wid(cid, sid)
    pltpu.sync_copy(srcp.at[w], src_v)
    pltpu.sync_copy(dstp.at[w], dst_v)

    def step(j, _):
        pltpu.sync_copy(g_sh.at[src_v.at[j]], rows_v)
        pltpu.sync_copy(rows_v, acc_sh.at[dst_v.at[j]], add=True)
        return ()
    lax.fori_loop(0, _NCHUNK, step, ())
    plsc.subcore_barrier()

    pltpu.sync_copy(acc_sh.at[pl.ds(sid * _RPT, _RPT)],
                    out.at[cid, pl.ds(sid * _RPT, _RPT)])


def _conv_sc(g, srcp, dstp):
    mesh = plsc.VectorSubcoreMesh(core_axis_name="c", subcore_axis_name="s")
    return pl.kernel(
        _conv_body,
        out_type=jax.ShapeDtypeStruct((_NC, _NPAD, 64), jnp.bfloat16),
        mesh=mesh,
        compiler_params=pltpu.CompilerParams(use_tc_tiling_on_sc=False),
        scratch_types=[
            pltpu.VMEM((_NCHUNK, _CH), jnp.int32),
            pltpu.VMEM((_NCHUNK, _CH), jnp.int32),
            pltpu.VMEM((_CH, 64), jnp.bfloat16),
            pltpu.VMEM((64, 64), jnp.bfloat16),
            pltpu.VMEM_SHARED((_NPAD, 64), jnp.bfloat16),
            pltpu.VMEM_SHARED((_NPAD, 64), jnp.bfloat16),
        ],
    )(g, srcp, dstp)


def _dinv(dpt_ref):
    deg = dpt_ref[:, 0:1] + dpt_ref[:, 1:2] + 1.0
    return lax.rsqrt(deg)


def _mm1_body(x_ref, w_ref, dpt_ref, g_ref):
    h = jnp.dot(x_ref[...], w_ref[...], preferred_element_type=jnp.float32)
    g_ref[...] = (_dinv(dpt_ref) * h).astype(jnp.bfloat16)


def _mid_body(ap_ref, g1_ref, dpt_ref, b1_ref, w2_ref, g2_ref):
    dinv = _dinv(dpt_ref)
    acc = (ap_ref[0] + ap_ref[1]).astype(jnp.float32) \
        + g1_ref[...].astype(jnp.float32)
    h1 = jnp.maximum(dinv * acc + b1_ref[...], 0.0)
    g2_ref[...] = (dinv * jnp.dot(h1, w2_ref[...],
                                  preferred_element_type=jnp.float32)
                   ).astype(jnp.bfloat16)


def _pool_body(ap_ref, g2_ref, dpt_ref, b2_ref, bat_ref, out_ref):
    i = pl.program_id(0)
    dinv = _dinv(dpt_ref)
    h2 = dinv * ((ap_ref[0] + ap_ref[1]).astype(jnp.float32)
                 + g2_ref[...].astype(jnp.float32)) + b2_ref[...]
    ids = jax.lax.broadcasted_iota(jnp.int32, (_G, _BLK), 0)
    oht = (ids == bat_ref[0]).astype(jnp.float32)
    part = jnp.dot(oht, h2, preferred_element_type=jnp.float32)

    @pl.when(i == 0)
    def _():
        out_ref[...] = part

    @pl.when(i > 0)
    def _():
        out_ref[...] += part


def _mm1_tc(xp, W1, dpt):
    return pl.pallas_call(
        _mm1_body,
        grid=(_GRID,),
        in_specs=[pl.BlockSpec((_BLK, 128), lambda i: (i, 0)),
                  pl.BlockSpec((128, 64), lambda i: (0, 0)),
                  pl.BlockSpec((_BLK, _NC), lambda i: (i, 0))],
        out_specs=pl.BlockSpec((_BLK, 64), lambda i: (i, 0)),
        out_shape=jax.ShapeDtypeStruct((_NPAD, 64), jnp.bfloat16),
    )(xp, W1, dpt)


def _mid_tc(ap, g1, dpt, b1, W2):
    return pl.pallas_call(
        _mid_body,
        grid=(_GRID,),
        in_specs=[pl.BlockSpec((_NC, _BLK, 64), lambda i: (0, i, 0)),
                  pl.BlockSpec((_BLK, 64), lambda i: (i, 0)),
                  pl.BlockSpec((_BLK, _NC), lambda i: (i, 0)),
                  pl.BlockSpec((1, 64), lambda i: (0, 0)),
                  pl.BlockSpec((64, 64), lambda i: (0, 0))],
        out_specs=pl.BlockSpec((_BLK, 64), lambda i: (i, 0)),
        out_shape=jax.ShapeDtypeStruct((_NPAD, 64), jnp.bfloat16),
    )(ap, g1, dpt, b1, W2)


def _pool_tc(ap, g2, dpt, b2, bat3):
    return pl.pallas_call(
        _pool_body,
        grid=(_GRID,),
        in_specs=[pl.BlockSpec((_NC, _BLK, 64), lambda i: (0, i, 0)),
                  pl.BlockSpec((_BLK, 64), lambda i: (i, 0)),
                  pl.BlockSpec((_BLK, _NC), lambda i: (i, 0)),
                  pl.BlockSpec((1, 64), lambda i: (0, 0)),
                  pl.BlockSpec((1, 1, _BLK), lambda i: (i, 0, 0))],
        out_specs=pl.BlockSpec((_G, 64), lambda i: (0, 0)),
        out_shape=jax.ShapeDtypeStruct((_G, 64), jnp.float32),
    )(ap, g2, dpt, b2, bat3)


def kernel(x, edge_index, batch, W1, b1, W2, b2):
    src, dst = edge_index[0], edge_index[1]
    epw = _E // _NW
    pad = _EPW_PAD - epw
    srcp = jnp.pad(src.reshape(_NW, epw), ((0, 0), (0, pad))
                   ).reshape(_NW, _NCHUNK, _CH)
    dstf = jnp.pad(dst.reshape(_NW, epw), ((0, 0), (0, pad)),
                   constant_values=_N)
    dstp = dstf.reshape(_NW, _NCHUNK, _CH)

    xp = jnp.pad(x, ((0, _NPAD - _N), (0, 0)))
    bat3 = jnp.pad(batch, (0, _NPAD - _N),
                   constant_values=_G).reshape(_GRID, 1, _BLK)

    dp = _deg_sc(dstf)
    dpt = dp.T

    g1 = _mm1_tc(xp, W1, dpt)
    ap1 = _conv_sc(g1, srcp, dstp)
    g2 = _mid_tc(ap1, g1, dpt, b1.reshape(1, 64), W2)
    ap2 = _conv_sc(g2, srcp, dstp)
    out = _pool_tc(ap2, g2, dpt, b2.reshape(1, 64), bat3)
    return out


# speedup vs baseline: 2.6786x; 1.0974x over previous
import functools

import jax
import jax.numpy as jnp
from jax import lax
from jax.experimental import pallas as pl
from jax.experimental.pallas import tpu as pltpu
from jax.experimental.pallas import tpu_sc as plsc

_N = 10000
_E = 320000
_G = 64

_NC = 2
_NS = 16
_NW = _NC * _NS

_EPW = _E // _NW
_CH = 128
_NFULL = _EPW // _CH
_TAIL = _EPW - _NFULL * _CH

_NPAD = 10240
_RPT = _NPAD // _NS
_SPT = _N // _NS
_BLK = 2000
_GRID = _N // _BLK


def _wid(cid, sid):
    return cid * _NS + sid


def _deg_body(ei3, out, dst_v, ones_v, zbuf_v, deg_sh):
    cid = lax.axis_index("c")
    sid = lax.axis_index("s")

    ones16 = jnp.full((16,), 1.0, jnp.float32)

    def ob(i, _):
        ones_v[pl.ds(i * 16, 16)] = ones16
        return ()
    lax.fori_loop(0, _EPW // 16, ob, ())
    zeros16 = jnp.zeros((16,), jnp.float32)

    def zb(i, _):
        zbuf_v[pl.ds(i * 16, 16)] = zeros16
        return ()
    lax.fori_loop(0, _RPT // 16, zb, ())
    pltpu.sync_copy(zbuf_v, deg_sh.at[pl.ds(sid * _RPT, _RPT)])
    plsc.subcore_barrier()

    pltpu.sync_copy(ei3.at[1, _wid(cid, sid)], dst_v)
    pltpu.sync_copy(ones_v, deg_sh.at[dst_v], add=True)
    plsc.subcore_barrier()

    pltpu.sync_copy(deg_sh.at[pl.ds(sid * _RPT, _RPT)],
                    out.at[cid, pl.ds(sid * _RPT, _RPT)])


def _deg_sc(ei3):
    mesh = plsc.VectorSubcoreMesh(core_axis_name="c", subcore_axis_name="s")
    return pl.kernel(
        _deg_body,
        out_type=jax.ShapeDtypeStruct((_NC, _NPAD), jnp.float32),
        mesh=mesh,
        compiler_params=pltpu.CompilerParams(use_tc_tiling_on_sc=False),
        scratch_types=[
            pltpu.VMEM((_EPW,), jnp.int32),
            pltpu.VMEM((_EPW,), jnp.float32),
            pltpu.VMEM((_RPT,), jnp.float32),
            pltpu.VMEM_SHARED((_NPAD,), jnp.float32),
        ],
    )(ei3)


def _conv_body(g, ei3, out, src_v, dst_v, rows_v, zbuf_v, acc_sh, g_sh):
    cid = lax.axis_index("c")
    sid = lax.axis_index("s")

    zeros32 = jnp.zeros((32,), jnp.bfloat16)

    def zb(i, _):
        for j in range(2):
            zbuf_v[i, pl.ds(j * 32, 32)] = zeros32
        return ()
    lax.fori_loop(0, 64, zb, ())
    pltpu.sync_copy(g.at[pl.ds(sid * _SPT, _SPT)],
                    g_sh.at[pl.ds(sid * _SPT, _SPT)])
    for r in range(_RPT // 64):
        pltpu.sync_copy(zbuf_v, acc_sh.at[pl.ds(sid * _RPT + r * 64, 64)])
    plsc.subcore_barrier()

    w = _wid(cid, sid)
    pltpu.sync_copy(ei3.at[0, w], src_v)
    pltpu.sync_copy(ei3.at[1, w], dst_v)

    def step(j, _):
        pltpu.sync_copy(g_sh.at[src_v.at[pl.ds(j * _CH, _CH)]], rows_v)
        pltpu.sync_copy(rows_v, acc_sh.at[dst_v.at[pl.ds(j * _CH, _CH)]],
                        add=True)
        return ()
    lax.fori_loop(0, _NFULL, step, ())
    pltpu.sync_copy(g_sh.at[src_v.at[pl.ds(_NFULL * _CH, _TAIL)]],
                    rows_v.at[pl.ds(0, _TAIL)])
    pltpu.sync_copy(rows_v.at[pl.ds(0, _TAIL)],
                    acc_sh.at[dst_v.at[pl.ds(_NFULL * _CH, _TAIL)]],
                    add=True)
    plsc.subcore_barrier()

    pltpu.sync_copy(acc_sh.at[pl.ds(sid * _RPT, _RPT)],
                    out.at[cid, pl.ds(sid * _RPT, _RPT)])


def _conv_sc(g, ei3):
    mesh = plsc.VectorSubcoreMesh(core_axis_name="c", subcore_axis_name="s")
    return pl.kernel(
        _conv_body,
        out_type=jax.ShapeDtypeStruct((_NC, _NPAD, 64), jnp.bfloat16),
        mesh=mesh,
        compiler_params=pltpu.CompilerParams(use_tc_tiling_on_sc=False),
        scratch_types=[
            pltpu.VMEM((_EPW,), jnp.int32),
            pltpu.VMEM((_EPW,), jnp.int32),
            pltpu.VMEM((_CH, 64), jnp.bfloat16),
            pltpu.VMEM((64, 64), jnp.bfloat16),
            pltpu.VMEM_SHARED((_NPAD, 64), jnp.bfloat16),
            pltpu.VMEM_SHARED((_NPAD, 64), jnp.bfloat16),
        ],
    )(g, ei3)


def _dinv(dpt_ref):
    deg = dpt_ref[:, 0:1] + dpt_ref[:, 1:2] + 1.0
    return lax.rsqrt(deg)


def _mm1_body(x_ref, w_ref, dpt_ref, g_ref):
    h = jnp.dot(x_ref[...], w_ref[...], preferred_element_type=jnp.float32)
    g_ref[...] = (_dinv(dpt_ref) * h).astype(jnp.bfloat16)


def _mid_body(ap_ref, g1_ref, dpt_ref, b1_ref, w2_ref, g2_ref):
    dinv = _dinv(dpt_ref)
    acc = (ap_ref[0] + ap_ref[1]).astype(jnp.float32) \
        + g1_ref[...].astype(jnp.float32)
    h1 = jnp.maximum(dinv * acc + b1_ref[...], 0.0)
    g2_ref[...] = (dinv * jnp.dot(h1, w2_ref[...],
                                  preferred_element_type=jnp.float32)
                   ).astype(jnp.bfloat16)


def _pool_body(ap_ref, g2_ref, dpt_ref, b2_ref, bat_ref, out_ref):
    i = pl.program_id(0)
    dinv = _dinv(dpt_ref)
    h2 = dinv * ((ap_ref[0] + ap_ref[1]).astype(jnp.float32)
                 + g2_ref[...].astype(jnp.float32)) + b2_ref[...]
    ids = jax.lax.broadcasted_iota(jnp.int32, (_G, _BLK), 0)
    oht = (ids == bat_ref[0]).astype(jnp.float32)
    part = jnp.dot(oht, h2, preferred_element_type=jnp.float32)

    @pl.when(i == 0)
    def _():
        out_ref[...] = part

    @pl.when(i > 0)
    def _():
        out_ref[...] += part


def _mm1_tc(x, W1, dpt):
    return pl.pallas_call(
        _mm1_body,
        grid=(_GRID,),
        in_specs=[pl.BlockSpec((_BLK, 128), lambda i: (i, 0)),
                  pl.BlockSpec((128, 64), lambda i: (0, 0)),
                  pl.BlockSpec((_BLK, _NC), lambda i: (i, 0))],
        out_specs=pl.BlockSpec((_BLK, 64), lambda i: (i, 0)),
        out_shape=jax.ShapeDtypeStruct((_N, 64), jnp.bfloat16),
    )(x, W1, dpt)


def _mid_tc(ap, g1, dpt, b1, W2):
    return pl.pallas_call(
        _mid_body,
        grid=(_GRID,),
        in_specs=[pl.BlockSpec((_NC, _BLK, 64), lambda i: (0, i, 0)),
                  pl.BlockSpec((_BLK, 64), lambda i: (i, 0)),
                  pl.BlockSpec((_BLK, _NC), lambda i: (i, 0)),
                  pl.BlockSpec((1, 64), lambda i: (0, 0)),
                  pl.BlockSpec((64, 64), lambda i: (0, 0))],
        out_specs=pl.BlockSpec((_BLK, 64), lambda i: (i, 0)),
        out_shape=jax.ShapeDtypeStruct((_N, 64), jnp.bfloat16),
    )(ap, g1, dpt, b1, W2)


def _pool_tc(ap, g2, dpt, b2, bat3):
    return pl.pallas_call(
        _pool_body,
        grid=(_GRID,),
        in_specs=[pl.BlockSpec((_NC, _BLK, 64), lambda i: (0, i, 0)),
                  pl.BlockSpec((_BLK, 64), lambda i: (i, 0)),
                  pl.BlockSpec((_BLK, _NC), lambda i: (i, 0)),
                  pl.BlockSpec((1, 64), lambda i: (0, 0)),
                  pl.BlockSpec((1, 1, _BLK), lambda i: (i, 0, 0))],
        out_specs=pl.BlockSpec((_G, 64), lambda i: (0, 0)),
        out_shape=jax.ShapeDtypeStruct((_G, 64), jnp.float32),
    )(ap, g2, dpt, b2, bat3)


def kernel(x, edge_index, batch, W1, b1, W2, b2):
    ei3 = edge_index.reshape(2, _NW, _EPW)
    bat3 = batch.reshape(_GRID, 1, _BLK)

    dp = _deg_sc(ei3)
    dpt = dp.T

    g1 = _mm1_tc(x, W1, dpt)
    ap1 = _conv_sc(g1, ei3)
    g2 = _mid_tc(ap1, g1, dpt, b1.reshape(1, 64), W2)
    ap2 = _conv_sc(g2, ei3)
    out = _pool_tc(ap2, g2, dpt, b2.reshape(1, 64), bat3)
    return out
